# baseline (speedup 1.0000x reference)
import jax
import jax.numpy as jnp
from jax import lax
from jax.experimental import pallas as pl
from jax.experimental.pallas import tpu as pltpu

_ALPHA = 3.0
_KTOP = 20


def _main_body(xt_ref, a0_ref, a1_ref, nv1_ref, nv2_ref,
               c1w_ref, b1_ref, c2w_ref, b2_ref,
               out1_ref, out2_ref):
    for s in range(xt_ref.shape[0]):
        _one_step(xt_ref[s], a0_ref, a1_ref, nv1_ref, nv2_ref,
                  c1w_ref, b1_ref, c2w_ref, b2_ref,
                  out1_ref.at[s], out2_ref.at[s])


def _one_step(xbl, a0_ref, a1_ref, nv1_ref, nv2_ref,
              c1w_ref, b1_ref, c2w_ref, b2_ref,
              out1_ref, out2_ref):
    a0 = a0_ref[...]
    a1 = a1_ref[...]
    n = a0.shape[0]

    x1 = lax.dot_general(xbl, a0, (((1,), (0,)), ((), ())))
    x2 = lax.dot_general(x1, a0, (((1,), (0,)), ((), ())))
    h1 = jnp.concatenate([xbl, x1, x2], axis=0)
    f1 = lax.dot_general(h1, c1w_ref[...], (((0,), (1,)), ((), ()))) + b1_ref[...]
    y1 = lax.dot_general(xbl, a1, (((1,), (0,)), ((), ())))
    y2 = lax.dot_general(y1, a1, (((1,), (0,)), ((), ())))
    h2 = jnp.concatenate([xbl, y1, y2], axis=0)
    f2 = lax.dot_general(h2, c2w_ref[...], (((0,), (1,)), ((), ()))) + b2_ref[...]

    nv1b = jnp.tanh(_ALPHA * (nv1_ref[...] * f1))
    nv2b = jnp.tanh(_ALPHA * (nv2_ref[...] * f2))
    logits = (lax.dot_general(nv1b, nv2b, (((1,), (1,)), ((), ())))
              - lax.dot_general(nv2b, nv1b, (((1,), (1,)), ((), ()))))
    adj = jnp.maximum(jnp.tanh(_ALPHA * logits), 0.0)

    t = jnp.max(adj, axis=1, keepdims=True)
    for _ in range(_KTOP - 1):
        sel = jnp.where(adj < t, adj, -1.0)
        c = -jnp.sum(jnp.minimum(sel, 0.0), axis=1, keepdims=True)
        m = jnp.max(sel, axis=1, keepdims=True)
        t = jnp.where(c >= float(_KTOP), t, m)

    gt_mask = adj > t
    ngt = jnp.sum(jnp.where(gt_mask, 1.0, 0.0), axis=1, keepdims=True)
    tie = adj == t
    tief = jnp.where(tie, 1.0, 0.0).astype(jnp.bfloat16)
    ii = lax.broadcasted_iota(jnp.int32, (n, n), 0)
    jj = lax.broadcasted_iota(jnp.int32, (n, n), 1)
    ltm = jnp.where(ii < jj, 1.0, 0.0).astype(jnp.bfloat16)
    rank = lax.dot_general(tief, ltm, (((1,), (0,)), ((), ())),
                           preferred_element_type=jnp.float32)
    keep = jnp.logical_or(gt_mask,
                          jnp.logical_and(tie, rank < (float(_KTOP) - ngt)))
    adjm = jnp.where(keep, adj, 0.0)

    eye = jnp.where(ii == jj, 1.0, 0.0)
    adj1 = adjm + eye
    rs = jnp.sum(adj1, axis=1, keepdims=True)
    adjp = adj1 / rs
    out1_ref[...] = adjp
    adjp_t = adjp.T
    cs_t = jnp.sum(adjp_t, axis=1, keepdims=True)
    out2_ref[...] = (adjp_t + eye) / (cs_t + 1.0)


def kernel(x, idx, emb1_w, emb2_w, lin1_w, lin1_b, lin2_w, lin2_b,
           conv1_w, conv1_b, conv2_w, conv2_b):
    b, c, n, l = x.shape
    f32 = jnp.float32

    nv1 = jnp.tanh(_ALPHA * (emb1_w[idx] @ lin1_w.T + lin1_b))
    nv2 = jnp.tanh(_ALPHA * (emb2_w[idx] @ lin2_w.T + lin2_b))
    d2 = jnp.sum((nv1[:, None, :] - nv2[None, :, :]) ** 2, axis=-1)
    g = jnp.exp(-d2 / (2.0 * _ALPHA ** 2))
    g = lax.stop_gradient(g)
    rowsum0 = jnp.sum(g, axis=1)
    a0 = jnp.where(rowsum0 > 0, 1.0 / rowsum0, 0.0)[:, None] * g
    gt = g.T
    rowsum1 = jnp.sum(gt, axis=1)
    a1 = jnp.where(rowsum1 > 0, 1.0 / rowsum1, 0.0)[:, None] * gt

    b1 = conv1_b.reshape(1, c)
    b2 = conv2_b.reshape(1, c)
    xt = x.transpose(0, 3, 1, 2)

    rep = lambda shape: pl.BlockSpec(shape, lambda bb, ll: tuple(0 for _ in shape))
    lsub = 12
    adjp, adjt = pl.pallas_call(
        _main_body,
        grid=(b, l // lsub),
        in_specs=[pl.BlockSpec((None, lsub, c, n), lambda bb, ll: (bb, ll, 0, 0)),
                  rep((n, n)), rep((n, n)), rep((n, c)), rep((n, c)),
                  rep((c, 3 * c)), rep((1, c)), rep((c, 3 * c)), rep((1, c))],
        out_specs=[pl.BlockSpec((None, lsub, n, n), lambda bb, ll: (bb, ll, 0, 0)),
                   pl.BlockSpec((None, lsub, n, n), lambda bb, ll: (bb, ll, 0, 0))],
        out_shape=[jax.ShapeDtypeStruct((b, l, n, n), f32),
                   jax.ShapeDtypeStruct((b, l, n, n), f32)],
        compiler_params=pltpu.CompilerParams(
            dimension_semantics=("parallel", "parallel")),
    )(xt, a0, a1, nv1, nv2, conv1_w, b1, conv2_w, b2)

    return (adjp, adjt)


# speedup vs baseline: 1.1818x; 1.1818x over previous
import jax
import jax.numpy as jnp
from jax import lax
from jax.experimental import pallas as pl
from jax.experimental.pallas import tpu as pltpu

_ALPHA = 3.0
_KTOP = 20


def _main_body(xt_ref, a0_ref, a1_ref, nv1_ref, nv2_ref,
               c1w_ref, b1_ref, c2w_ref, b2_ref,
               out1_ref, out2_ref):
    for s in range(xt_ref.shape[0]):
        _one_step(xt_ref[s], a0_ref, a1_ref, nv1_ref, nv2_ref,
                  c1w_ref, b1_ref, c2w_ref, b2_ref,
                  out1_ref.at[s], out2_ref.at[s])


def _one_step(xbl, a0_ref, a1_ref, nv1_ref, nv2_ref,
              c1w_ref, b1_ref, c2w_ref, b2_ref,
              out1_ref, out2_ref):
    a0 = a0_ref[...]
    a1 = a1_ref[...]
    n = a0.shape[0]

    x1 = lax.dot_general(xbl, a0, (((1,), (0,)), ((), ())))
    x2 = lax.dot_general(x1, a0, (((1,), (0,)), ((), ())))
    h1 = jnp.concatenate([xbl, x1, x2], axis=0)
    f1 = lax.dot_general(h1, c1w_ref[...], (((0,), (1,)), ((), ()))) + b1_ref[...]
    y1 = lax.dot_general(xbl, a1, (((1,), (0,)), ((), ())))
    y2 = lax.dot_general(y1, a1, (((1,), (0,)), ((), ())))
    h2 = jnp.concatenate([xbl, y1, y2], axis=0)
    f2 = lax.dot_general(h2, c2w_ref[...], (((0,), (1,)), ((), ()))) + b2_ref[...]

    nv1b = jnp.tanh(_ALPHA * (nv1_ref[...] * f1))
    nv2b = jnp.tanh(_ALPHA * (nv2_ref[...] * f2))
    logits = (lax.dot_general(nv1b, nv2b, (((1,), (1,)), ((), ())))
              - lax.dot_general(nv2b, nv1b, (((1,), (1,)), ((), ()))))
    adj = jnp.maximum(jnp.tanh(_ALPHA * logits), 0.0)

    t = jnp.max(adj, axis=1, keepdims=True)
    for _ in range(_KTOP - 1):
        sel = jnp.where(adj < t, adj, -1.0)
        c = -jnp.sum(jnp.minimum(sel, 0.0), axis=1, keepdims=True)
        m = jnp.max(sel, axis=1, keepdims=True)
        t = jnp.where(c >= float(_KTOP), t, m)

    gt_mask = adj > t
    ngt = jnp.sum(jnp.where(gt_mask, 1.0, 0.0), axis=1, keepdims=True)
    tie = adj == t
    tief = jnp.where(tie, 1.0, 0.0).astype(jnp.bfloat16)
    ii = lax.broadcasted_iota(jnp.int32, (n, n), 0)
    jj = lax.broadcasted_iota(jnp.int32, (n, n), 1)
    ltm = jnp.where(ii < jj, 1.0, 0.0).astype(jnp.bfloat16)
    rank = lax.dot_general(tief, ltm, (((1,), (0,)), ((), ())),
                           preferred_element_type=jnp.float32)
    keep = jnp.logical_or(gt_mask,
                          jnp.logical_and(tie, rank < (float(_KTOP) - ngt)))
    adjm = jnp.where(keep, adj, 0.0)

    eye = jnp.where(ii == jj, 1.0, 0.0)
    adj1 = adjm + eye
    rs = jnp.sum(adj1, axis=1, keepdims=True)
    adjp = adj1 / rs
    out1_ref[...] = adjp
    adjp_t = adjp.T
    cs_t = jnp.sum(adjp_t, axis=1, keepdims=True)
    out2_ref[...] = (adjp_t + eye) / (cs_t + 1.0)


def kernel(x, idx, emb1_w, emb2_w, lin1_w, lin1_b, lin2_w, lin2_b,
           conv1_w, conv1_b, conv2_w, conv2_b):
    b, c, n, l = x.shape
    f32 = jnp.float32

    nv1 = jnp.tanh(_ALPHA * (emb1_w[idx] @ lin1_w.T + lin1_b))
    nv2 = jnp.tanh(_ALPHA * (emb2_w[idx] @ lin2_w.T + lin2_b))
    d2 = jnp.sum((nv1[:, None, :] - nv2[None, :, :]) ** 2, axis=-1)
    g = jnp.exp(-d2 / (2.0 * _ALPHA ** 2))
    g = lax.stop_gradient(g)
    rowsum0 = jnp.sum(g, axis=1)
    a0 = jnp.where(rowsum0 > 0, 1.0 / rowsum0, 0.0)[:, None] * g
    gt = g.T
    rowsum1 = jnp.sum(gt, axis=1)
    a1 = jnp.where(rowsum1 > 0, 1.0 / rowsum1, 0.0)[:, None] * gt

    b1 = conv1_b.reshape(1, c)
    b2 = conv2_b.reshape(1, c)
    xt = x.transpose(0, 3, 1, 2)

    rep = lambda shape: pl.BlockSpec(shape, lambda bb, ll: tuple(0 for _ in shape))
    lsub = 4
    adjp, adjt = pl.pallas_call(
        _main_body,
        grid=(b, l // lsub),
        in_specs=[pl.BlockSpec((None, lsub, c, n), lambda bb, ll: (bb, ll, 0, 0)),
                  rep((n, n)), rep((n, n)), rep((n, c)), rep((n, c)),
                  rep((c, 3 * c)), rep((1, c)), rep((c, 3 * c)), rep((1, c))],
        out_specs=[pl.BlockSpec((None, lsub, n, n), lambda bb, ll: (bb, ll, 0, 0)),
                   pl.BlockSpec((None, lsub, n, n), lambda bb, ll: (bb, ll, 0, 0))],
        out_shape=[jax.ShapeDtypeStruct((b, l, n, n), f32),
                   jax.ShapeDtypeStruct((b, l, n, n), f32)],
        compiler_params=pltpu.CompilerParams(
            dimension_semantics=("parallel", "parallel")),
    )(xt, a0, a1, nv1, nv2, conv1_w, b1, conv2_w, b2)

    return (adjp, adjt)


# speedup vs baseline: 1.1938x; 1.0101x over previous
import jax
import jax.numpy as jnp
from jax import lax
from jax.experimental import pallas as pl
from jax.experimental.pallas import tpu as pltpu

_ALPHA = 3.0
_KTOP = 20


def _main_body(xt_ref, a0_ref, a1_ref, nv1_ref, nv2_ref,
               c1w_ref, b1_ref, c2w_ref, b2_ref,
               out1_ref, out2_ref):
    for s in range(xt_ref.shape[0]):
        _one_step(xt_ref[s], a0_ref, a1_ref, nv1_ref, nv2_ref,
                  c1w_ref, b1_ref, c2w_ref, b2_ref,
                  out1_ref.at[s], out2_ref.at[s])


def _one_step(xbl, a0_ref, a1_ref, nv1_ref, nv2_ref,
              c1w_ref, b1_ref, c2w_ref, b2_ref,
              out1_ref, out2_ref):
    a0 = a0_ref[...]
    a1 = a1_ref[...]
    n = a0.shape[0]

    x1 = lax.dot_general(xbl, a0, (((1,), (0,)), ((), ())))
    x2 = lax.dot_general(x1, a0, (((1,), (0,)), ((), ())))
    h1 = jnp.concatenate([xbl, x1, x2], axis=0)
    f1 = lax.dot_general(h1, c1w_ref[...], (((0,), (1,)), ((), ()))) + b1_ref[...]
    y1 = lax.dot_general(xbl, a1, (((1,), (0,)), ((), ())))
    y2 = lax.dot_general(y1, a1, (((1,), (0,)), ((), ())))
    h2 = jnp.concatenate([xbl, y1, y2], axis=0)
    f2 = lax.dot_general(h2, c2w_ref[...], (((0,), (1,)), ((), ()))) + b2_ref[...]

    nv1b = jnp.tanh(_ALPHA * (nv1_ref[...] * f1))
    nv2b = jnp.tanh(_ALPHA * (nv2_ref[...] * f2))
    logits = (lax.dot_general(nv1b, nv2b, (((1,), (1,)), ((), ())))
              - lax.dot_general(nv2b, nv1b, (((1,), (1,)), ((), ()))))
    adj = jnp.maximum(jnp.tanh(_ALPHA * logits), 0.0)

    t = jnp.max(adj, axis=1, keepdims=True)
    for _ in range(_KTOP - 1):
        sel = jnp.where(adj < t, adj, -1.0)
        c = -jnp.sum(jnp.minimum(sel, 0.0), axis=1, keepdims=True)
        m = jnp.max(sel, axis=1, keepdims=True)
        t = jnp.where(c >= float(_KTOP), t, m)

    gt_mask = adj > t
    ngt = jnp.sum(jnp.where(gt_mask, 1.0, 0.0), axis=1, keepdims=True)
    tie = adj == t
    tief = jnp.where(tie, 1.0, 0.0).astype(jnp.bfloat16)
    ii = lax.broadcasted_iota(jnp.int32, (n, n), 0)
    jj = lax.broadcasted_iota(jnp.int32, (n, n), 1)
    ltm = jnp.where(ii < jj, 1.0, 0.0).astype(jnp.bfloat16)
    rank = lax.dot_general(tief, ltm, (((1,), (0,)), ((), ())),
                           preferred_element_type=jnp.float32)
    keep = jnp.logical_or(gt_mask,
                          jnp.logical_and(tie, rank < (float(_KTOP) - ngt)))
    adjm = jnp.where(keep, adj, 0.0)

    eye = jnp.where(ii == jj, 1.0, 0.0)
    adj1 = adjm + eye
    rs = jnp.sum(adj1, axis=1, keepdims=True)
    adjp = adj1 / rs
    out1_ref[...] = adjp
    adjp_t = adjp.T
    cs_t = jnp.sum(adjp_t, axis=1, keepdims=True)
    out2_ref[...] = (adjp_t + eye) / (cs_t + 1.0)


def kernel(x, idx, emb1_w, emb2_w, lin1_w, lin1_b, lin2_w, lin2_b,
           conv1_w, conv1_b, conv2_w, conv2_b):
    b, c, n, l = x.shape
    f32 = jnp.float32

    nv1 = jnp.tanh(_ALPHA * (emb1_w[idx] @ lin1_w.T + lin1_b))
    nv2 = jnp.tanh(_ALPHA * (emb2_w[idx] @ lin2_w.T + lin2_b))
    d2 = jnp.sum((nv1[:, None, :] - nv2[None, :, :]) ** 2, axis=-1)
    g = jnp.exp(-d2 / (2.0 * _ALPHA ** 2))
    g = lax.stop_gradient(g)
    rowsum0 = jnp.sum(g, axis=1)
    a0 = jnp.where(rowsum0 > 0, 1.0 / rowsum0, 0.0)[:, None] * g
    gt = g.T
    rowsum1 = jnp.sum(gt, axis=1)
    a1 = jnp.where(rowsum1 > 0, 1.0 / rowsum1, 0.0)[:, None] * gt

    b1 = conv1_b.reshape(1, c)
    b2 = conv2_b.reshape(1, c)
    xt = x.transpose(0, 3, 1, 2)

    rep = lambda shape: pl.BlockSpec(shape, lambda bb, ll: tuple(0 for _ in shape))
    lsub = 6
    adjp, adjt = pl.pallas_call(
        _main_body,
        grid=(b, l // lsub),
        in_specs=[pl.BlockSpec((None, lsub, c, n), lambda bb, ll: (bb, ll, 0, 0)),
                  rep((n, n)), rep((n, n)), rep((n, c)), rep((n, c)),
                  rep((c, 3 * c)), rep((1, c)), rep((c, 3 * c)), rep((1, c))],
        out_specs=[pl.BlockSpec((None, lsub, n, n), lambda bb, ll: (bb, ll, 0, 0)),
                   pl.BlockSpec((None, lsub, n, n), lambda bb, ll: (bb, ll, 0, 0))],
        out_shape=[jax.ShapeDtypeStruct((b, l, n, n), f32),
                   jax.ShapeDtypeStruct((b, l, n, n), f32)],
        compiler_params=pltpu.CompilerParams(
            dimension_semantics=("parallel", "parallel"),
            allow_input_fusion=[True] + [False] * 8),
    )(xt, a0, a1, nv1, nv2, conv1_w, b1, conv2_w, b2)

    return (adjp, adjt)
